# baseline (speedup 1.0000x reference)
import functools

import jax
import jax.numpy as jnp
from jax import lax
from jax.experimental import pallas as pl
from jax.experimental.pallas import tpu as pltpu
from jax.experimental.pallas import tpu_sc as plsc

N_ROWS = 100000
DIM = 128

TC_BLK = 1024
TC_NG = 49
TC_ROWS = TC_BLK * TC_NG
SC_ROWS = N_ROWS - TC_ROWS

NC = 2
NS = 16
NW = NC * NS
SHARD = SC_ROWS // NW
CHUNK = 128
N_FULL = SHARD // CHUNK
TAIL = SHARD - N_FULL * CHUNK
TGRP = (TAIL + 15) // 16
NBUF = 3
L = 16
NF = DIM // L
CW = CHUNK * DIM
GPC = CHUNK // L
DP = (N_FULL * CHUNK + TGRP * L)
PW = DIM + L
NEG = -1e30

_PERM_DNUMS = lax.GatherDimensionNumbers(
    offset_dims=(), collapsed_slice_dims=(0,), start_index_map=(0,))


def _perm(v, perm_idx):
    return lax.gather(v, perm_idx[:, None], _PERM_DNUMS, slice_sizes=(1,),
                      mode=lax.GatherScatterMode.PROMISE_IN_BOUNDS)


def _butterfly_sum(vecs):
    idx = lax.iota(jnp.int32, L)
    for sh in (1, 2, 4, 8):
        pidx = jnp.bitwise_xor(idx, sh)
        low = (idx & sh) == 0
        nxt = []
        for k in range(0, len(vecs), 2):
            a, b = vecs[k], vecs[k + 1]
            nxt.append(jnp.where(low, a, _perm(b, pidx)) +
                       jnp.where(low, _perm(a, pidx), b))
        vecs = nxt
    return vecs[0]


def _lane_reduce(v, op):
    idx = lax.iota(jnp.int32, L)
    for sh in (8, 4, 2, 1):
        v = op(v, _perm(v, jnp.bitwise_xor(idx, sh)))
    return v


def _dist_group(kb, qs, boff):
    accs = []
    for j in range(L):
        off = boff + j * DIM
        a = jnp.abs(kb[pl.ds(off, L)] - qs[0])
        for f in range(1, NF):
            a = a + jnp.abs(kb[pl.ds(off + f * L, L)] - qs[f])
        accs.append(a)
    return -_butterfly_sum(accs)


def _wsum_group(vb, p16, boff, accs):
    out = list(accs)
    for j in range(L):
        pw = _perm(p16, jnp.full((L,), j, jnp.int32))
        off = boff + j * DIM
        for f in range(NF):
            out[f] = out[f] + pw * vb[pl.ds(off + f * L, L)]
    return tuple(out)


def _sc_body(q_hbm, keys_hbm, vals_hbm, part_out,
             q_v, buf0, buf1, buf2, d_v, p_v, out_v, sem0, sem1, sem2):
    cid = lax.axis_index("c")
    sid = lax.axis_index("s")
    wid = sid * NC + cid
    base = (TC_ROWS + wid * SHARD) * DIM

    zero16 = jnp.zeros((L,), jnp.float32)
    negvec = jnp.full((L,), NEG, jnp.float32)
    lane = lax.iota(jnp.int32, L)
    bufs = [buf0, buf1, buf2]
    sems = [sem0, sem1, sem2]

    def start(hbm, c, b):
        return pltpu.async_copy(hbm.at[pl.ds(base + c * CW, CW)],
                                bufs[b].at[pl.ds(0, CW)], sems[b])

    def wait(hbm, b):
        pltpu.make_async_copy(hbm.at[pl.ds(base, CW)],
                              bufs[b].at[pl.ds(0, CW)], sems[b]).wait()

    for b in range(NBUF):
        start(keys_hbm, b, b)
    pltpu.sync_copy(q_hbm, q_v)
    qs = [q_v[pl.ds(L * f, L)] for f in range(NF)]

    def tri_a(t, _):
        for b in range(NBUF):
            c = t * NBUF + b
            wait(keys_hbm, b)
            kb = bufs[b]

            def gbody(g, _, kb=kb, c=c):
                d16 = _dist_group(kb, qs, g * (L * DIM))
                d_v[pl.ds(c * CHUNK + g * L, L)] = d16
                return 0
            lax.fori_loop(0, GPC, gbody, 0)

            @pl.when(c + NBUF < N_FULL)
            def _(c=c, b=b):
                start(keys_hbm, c + NBUF, b)
        return 0
    lax.fori_loop(0, N_FULL // NBUF, tri_a, 0)

    pltpu.sync_copy(keys_hbm.at[pl.ds(base + N_FULL * CW, TAIL * DIM)],
                    buf0.at[pl.ds(0, TAIL * DIM)])

    def tail_a(g, _):
        d16 = _dist_group(buf0, qs, g * (L * DIM))
        valid = (g * L + lane) < TAIL
        d_v[pl.ds(N_FULL * CHUNK + g * L, L)] = jnp.where(valid, d16, negvec)
        return 0
    lax.fori_loop(0, TGRP, tail_a, 0)

    for i in range(TAIL * NF, TGRP * L * NF):
        buf0[pl.ds(i * L, L)] = zero16

    for b in range(NBUF):
        start(vals_hbm, b, b)

    def max_body(i, mv):
        return jnp.maximum(mv, d_v[pl.ds(i * L, L)])
    mvec = lax.fori_loop(0, DP // L, max_body, negvec, unroll=4)
    m = _lane_reduce(mvec, jnp.maximum)

    def exp_body(i, sv):
        pv = jnp.exp(d_v[pl.ds(i * L, L)] - m)
        p_v[pl.ds(i * L, L)] = pv
        return sv + pv
    svec = lax.fori_loop(0, DP // L, exp_body, zero16, unroll=4)
    s = _lane_reduce(svec, jnp.add)

    def tri_b(t, accs):
        for b in range(NBUF):
            c = t * NBUF + b
            wait(vals_hbm, b)
            vb = bufs[b]

            def gbody(g, a, vb=vb, c=c):
                p16 = p_v[pl.ds(c * CHUNK + g * L, L)]
                return _wsum_group(vb, p16, g * (L * DIM), a)
            accs = lax.fori_loop(0, GPC, gbody, accs)

            @pl.when(c + NBUF < N_FULL)
            def _(c=c, b=b):
                start(vals_hbm, c + NBUF, b)
        return accs
    accs = lax.fori_loop(0, N_FULL // NBUF, tri_b,
                         tuple(zero16 for _ in range(NF)))

    pltpu.sync_copy(vals_hbm.at[pl.ds(base + N_FULL * CW, TAIL * DIM)],
                    buf0.at[pl.ds(0, TAIL * DIM)])

    def tail_b(g, a):
        p16 = p_v[pl.ds(N_FULL * CHUNK + g * L, L)]
        return _wsum_group(buf0, p16, g * (L * DIM), a)
    accs = lax.fori_loop(0, TGRP, tail_b, accs)

    for f in range(NF):
        out_v[pl.ds(L * f, L)] = accs[f]
    tail_vec = jnp.where(lane == 0, m, jnp.where(lane == 1, s, 0.0))
    out_v[pl.ds(DIM, L)] = tail_vec
    pltpu.sync_copy(out_v, part_out.at[pl.ds(wid * PW, PW)])


def _tc_stream_body(q_ref, k_ref, v_ref, part_ref, m_ref, s_ref, acc_ref):
    g = pl.program_id(0)

    @pl.when(g == 0)
    def _():
        m_ref[0, 0] = NEG
        s_ref[0, 0] = 0.0
        acc_ref[...] = jnp.zeros((1, DIM), jnp.float32)

    ones_b = jnp.ones((1, DIM), jnp.bfloat16)
    ad = jnp.abs(k_ref[...] - q_ref[...])
    hi = ad.astype(jnp.bfloat16)
    lo = (ad - hi.astype(jnp.float32)).astype(jnp.bfloat16)
    cdn = (((1,), (1,)), ((), ()))
    d = -(lax.dot_general(ones_b, hi, cdn,
                          preferred_element_type=jnp.float32) +
          lax.dot_general(ones_b, lo, cdn,
                          preferred_element_type=jnp.float32))
    m_old = m_ref[0, 0]
    m_new = jnp.maximum(m_old, jnp.max(d))
    p = jnp.exp(d - m_new)
    scale = jnp.exp(m_old - m_new)
    m_ref[0, 0] = m_new
    s_ref[0, 0] = s_ref[0, 0] * scale + jnp.sum(p)
    acc_ref[...] = acc_ref[...] * scale + lax.dot_general(
        p, v_ref[...], (((1,), (0,)), ((), ())),
        precision=lax.Precision.HIGHEST,
        preferred_element_type=jnp.float32)

    @pl.when(g == TC_NG - 1)
    def _():
        lanei = lax.broadcasted_iota(jnp.int32, (1, L), 1)
        tail_vec = jnp.where(lanei == 0, m_ref[0, 0],
                             jnp.where(lanei == 1, s_ref[0, 0], 0.0))
        part_ref[...] = jnp.concatenate([acc_ref[...], tail_vec], axis=1)


def _combine_body(scp_ref, tcp_ref, out_ref):
    accs, ms, ss = [], [], []
    for w in range(NW):
        accs.append(scp_ref[pl.ds(w * PW, DIM)])
        t = scp_ref[pl.ds(w * PW + DIM, L)]
        ms.append(t[0:1])
        ss.append(t[1:2])
    tp = tcp_ref[...]
    accs.append(tp[0, 0:DIM])
    ms.append(tp[0, DIM:DIM + 1])
    ss.append(tp[0, DIM + 1:DIM + 2])
    gmax = ms[0]
    for mw in ms[1:]:
        gmax = jnp.maximum(gmax, mw)
    num = jnp.zeros((DIM,), jnp.float32)
    den = jnp.zeros((1,), jnp.float32)
    for mw, sw, aw in zip(ms, ss, accs):
        alpha = jnp.exp(mw - gmax)
        num = num + alpha * aw
        den = den + alpha * sw
    out_ref[...] = (num / den).reshape(1, DIM)


@jax.jit
def kernel(query, keys, values):
    mesh = plsc.VectorSubcoreMesh(core_axis_name="c", subcore_axis_name="s")
    sc_part = pl.kernel(
        _sc_body,
        mesh=mesh,
        compiler_params=pltpu.CompilerParams(needs_layout_passes=False),
        out_type=jax.ShapeDtypeStruct((NW * PW,), jnp.float32),
        scratch_types=[
            pltpu.VMEM((DIM,), jnp.float32),
            pltpu.VMEM((CW,), jnp.float32),
            pltpu.VMEM((CW,), jnp.float32),
            pltpu.VMEM((CW,), jnp.float32),
            pltpu.VMEM((DP,), jnp.float32),
            pltpu.VMEM((DP,), jnp.float32),
            pltpu.VMEM((PW,), jnp.float32),
            pltpu.SemaphoreType.DMA,
            pltpu.SemaphoreType.DMA,
            pltpu.SemaphoreType.DMA,
        ],
    )(query, keys.reshape(-1), values.reshape(-1))

    tc_part = pl.pallas_call(
        _tc_stream_body,
        grid=(TC_NG,),
        in_specs=[
            pl.BlockSpec((1, DIM), lambda g: (0, 0)),
            pl.BlockSpec((TC_BLK, DIM), lambda g: (g, 0)),
            pl.BlockSpec((TC_BLK, DIM), lambda g: (g, 0)),
        ],
        out_specs=pl.BlockSpec((1, PW), lambda g: (0, 0)),
        out_shape=jax.ShapeDtypeStruct((1, PW), jnp.float32),
        scratch_shapes=[
            pltpu.SMEM((1, 1), jnp.float32),
            pltpu.SMEM((1, 1), jnp.float32),
            pltpu.VMEM((1, DIM), jnp.float32),
        ],
    )(query.reshape(1, DIM), keys, values)

    out2d = pl.pallas_call(
        _combine_body,
        out_shape=jax.ShapeDtypeStruct((1, DIM), jnp.float32),
    )(sc_part, tc_part)
    return out2d.reshape(DIM)


# speedup vs baseline: 1.1877x; 1.1877x over previous
import functools

import jax
import jax.numpy as jnp
from jax import lax
from jax.experimental import pallas as pl
from jax.experimental.pallas import tpu as pltpu
from jax.experimental.pallas import tpu_sc as plsc

N_ROWS = 100000
DIM = 128

TC_BLK = 2048
TC_NG = 24
TC_ROWS = TC_BLK * TC_NG
SC_ROWS = N_ROWS - TC_ROWS

NC = 2
NS = 16
NW = NC * NS
SHARD = SC_ROWS // NW
CHUNK = 128
N_FULL = SHARD // CHUNK
TAIL = SHARD - N_FULL * CHUNK
TGRP = (TAIL + 15) // 16
NBUF = 3
L = 16
NF = DIM // L
CW = CHUNK * DIM
GPC = CHUNK // L
DP = (N_FULL * CHUNK + TGRP * L)
PW = DIM + L
NEG = -1e30

_PERM_DNUMS = lax.GatherDimensionNumbers(
    offset_dims=(), collapsed_slice_dims=(0,), start_index_map=(0,))


def _perm(v, perm_idx):
    return lax.gather(v, perm_idx[:, None], _PERM_DNUMS, slice_sizes=(1,),
                      mode=lax.GatherScatterMode.PROMISE_IN_BOUNDS)


def _butterfly_sum(vecs):
    idx = lax.iota(jnp.int32, L)
    for sh in (1, 2, 4, 8):
        pidx = jnp.bitwise_xor(idx, sh)
        low = (idx & sh) == 0
        nxt = []
        for k in range(0, len(vecs), 2):
            a, b = vecs[k], vecs[k + 1]
            nxt.append(jnp.where(low, a, _perm(b, pidx)) +
                       jnp.where(low, _perm(a, pidx), b))
        vecs = nxt
    return vecs[0]


def _lane_reduce(v, op):
    idx = lax.iota(jnp.int32, L)
    for sh in (8, 4, 2, 1):
        v = op(v, _perm(v, jnp.bitwise_xor(idx, sh)))
    return v


def _dist_group(kb, qs, boff):
    accs = []
    for j in range(L):
        off = boff + j * DIM
        a = jnp.abs(kb[pl.ds(off, L)] - qs[0])
        for f in range(1, NF):
            a = a + jnp.abs(kb[pl.ds(off + f * L, L)] - qs[f])
        accs.append(a)
    return -_butterfly_sum(accs)


def _wsum_group(vb, p16, boff, accs):
    out = list(accs)
    for j in range(L):
        pw = _perm(p16, jnp.full((L,), j, jnp.int32))
        off = boff + j * DIM
        for f in range(NF):
            out[f] = out[f] + pw * vb[pl.ds(off + f * L, L)]
    return tuple(out)


def _sc_body(q_hbm, keys_hbm, vals_hbm, part_out,
             q_v, buf0, buf1, buf2, d_v, p_v, out_v, sem0, sem1, sem2):
    cid = lax.axis_index("c")
    sid = lax.axis_index("s")
    wid = sid * NC + cid
    base = (TC_ROWS + wid * SHARD) * DIM

    zero16 = jnp.zeros((L,), jnp.float32)
    negvec = jnp.full((L,), NEG, jnp.float32)
    lane = lax.iota(jnp.int32, L)
    bufs = [buf0, buf1, buf2]
    sems = [sem0, sem1, sem2]

    def start(hbm, c, b):
        return pltpu.async_copy(hbm.at[pl.ds(base + c * CW, CW)],
                                bufs[b].at[pl.ds(0, CW)], sems[b])

    def wait(hbm, b):
        pltpu.make_async_copy(hbm.at[pl.ds(base, CW)],
                              bufs[b].at[pl.ds(0, CW)], sems[b]).wait()

    for b in range(NBUF):
        start(keys_hbm, b, b)
    pltpu.sync_copy(q_hbm, q_v)
    qs = [q_v[pl.ds(L * f, L)] for f in range(NF)]

    def tri_a(t, _):
        for b in range(NBUF):
            c = t * NBUF + b
            wait(keys_hbm, b)
            kb = bufs[b]

            def gbody(g, _, kb=kb, c=c):
                d16 = _dist_group(kb, qs, g * (L * DIM))
                d_v[pl.ds(c * CHUNK + g * L, L)] = d16
                return 0
            lax.fori_loop(0, GPC, gbody, 0)

            @pl.when(c + NBUF < N_FULL)
            def _(c=c, b=b):
                start(keys_hbm, c + NBUF, b)
        return 0
    lax.fori_loop(0, N_FULL // NBUF, tri_a, 0)

    pltpu.sync_copy(keys_hbm.at[pl.ds(base + N_FULL * CW, TAIL * DIM)],
                    buf0.at[pl.ds(0, TAIL * DIM)])

    def tail_a(g, _):
        d16 = _dist_group(buf0, qs, g * (L * DIM))
        valid = (g * L + lane) < TAIL
        d_v[pl.ds(N_FULL * CHUNK + g * L, L)] = jnp.where(valid, d16, negvec)
        return 0
    lax.fori_loop(0, TGRP, tail_a, 0)

    for i in range(TAIL * NF, TGRP * L * NF):
        buf0[pl.ds(i * L, L)] = zero16

    for b in range(NBUF):
        start(vals_hbm, b, b)

    def max_body(i, mv):
        return jnp.maximum(mv, d_v[pl.ds(i * L, L)])
    mvec = lax.fori_loop(0, DP // L, max_body, negvec, unroll=4)
    m = _lane_reduce(mvec, jnp.maximum)

    def exp_body(i, sv):
        pv = jnp.exp(d_v[pl.ds(i * L, L)] - m)
        p_v[pl.ds(i * L, L)] = pv
        return sv + pv
    svec = lax.fori_loop(0, DP // L, exp_body, zero16, unroll=4)
    s = _lane_reduce(svec, jnp.add)

    def tri_b(t, accs):
        for b in range(NBUF):
            c = t * NBUF + b
            wait(vals_hbm, b)
            vb = bufs[b]

            def gbody(g, a, vb=vb, c=c):
                p16 = p_v[pl.ds(c * CHUNK + g * L, L)]
                return _wsum_group(vb, p16, g * (L * DIM), a)
            accs = lax.fori_loop(0, GPC, gbody, accs)

            @pl.when(c + NBUF < N_FULL)
            def _(c=c, b=b):
                start(vals_hbm, c + NBUF, b)
        return accs
    accs = lax.fori_loop(0, N_FULL // NBUF, tri_b,
                         tuple(zero16 for _ in range(NF)))

    pltpu.sync_copy(vals_hbm.at[pl.ds(base + N_FULL * CW, TAIL * DIM)],
                    buf0.at[pl.ds(0, TAIL * DIM)])

    def tail_b(g, a):
        p16 = p_v[pl.ds(N_FULL * CHUNK + g * L, L)]
        return _wsum_group(buf0, p16, g * (L * DIM), a)
    accs = lax.fori_loop(0, TGRP, tail_b, accs)

    for f in range(NF):
        out_v[pl.ds(L * f, L)] = accs[f]
    tail_vec = jnp.where(lane == 0, m, jnp.where(lane == 1, s, 0.0))
    out_v[pl.ds(DIM, L)] = tail_vec
    pltpu.sync_copy(out_v, part_out.at[pl.ds(wid * PW, PW)])


def _tc_stream_body(q_ref, k_ref, v_ref, part_ref, m_ref, s_ref, acc_ref):
    g = pl.program_id(0)

    @pl.when(g == 0)
    def _():
        m_ref[0, 0] = NEG
        s_ref[0, 0] = 0.0
        acc_ref[...] = jnp.zeros((1, DIM), jnp.float32)

    ones_b = jnp.ones((1, DIM), jnp.bfloat16)
    ad = jnp.abs(k_ref[...] - q_ref[...])
    hi = ad.astype(jnp.bfloat16)
    lo = (ad - hi.astype(jnp.float32)).astype(jnp.bfloat16)
    cdn = (((1,), (1,)), ((), ()))
    d = -(lax.dot_general(ones_b, hi, cdn,
                          preferred_element_type=jnp.float32) +
          lax.dot_general(ones_b, lo, cdn,
                          preferred_element_type=jnp.float32))
    m_old = m_ref[0, 0]
    m_new = jnp.maximum(m_old, jnp.max(d))
    p = jnp.exp(d - m_new)
    scale = jnp.exp(m_old - m_new)
    m_ref[0, 0] = m_new
    s_ref[0, 0] = s_ref[0, 0] * scale + jnp.sum(p)
    acc_ref[...] = acc_ref[...] * scale + lax.dot_general(
        p, v_ref[...], (((1,), (0,)), ((), ())),
        precision=lax.Precision.HIGHEST,
        preferred_element_type=jnp.float32)

    @pl.when(g == TC_NG - 1)
    def _():
        lanei = lax.broadcasted_iota(jnp.int32, (1, L), 1)
        tail_vec = jnp.where(lanei == 0, m_ref[0, 0],
                             jnp.where(lanei == 1, s_ref[0, 0], 0.0))
        part_ref[...] = jnp.concatenate([acc_ref[...], tail_vec], axis=1)


def _combine_body(scp_ref, tcp_ref, out_ref):
    accs, ms, ss = [], [], []
    for w in range(NW):
        accs.append(scp_ref[pl.ds(w * PW, DIM)])
        t = scp_ref[pl.ds(w * PW + DIM, L)]
        ms.append(t[0:1])
        ss.append(t[1:2])
    tp = tcp_ref[...]
    accs.append(tp[0, 0:DIM])
    ms.append(tp[0, DIM:DIM + 1])
    ss.append(tp[0, DIM + 1:DIM + 2])
    gmax = ms[0]
    for mw in ms[1:]:
        gmax = jnp.maximum(gmax, mw)
    num = jnp.zeros((DIM,), jnp.float32)
    den = jnp.zeros((1,), jnp.float32)
    for mw, sw, aw in zip(ms, ss, accs):
        alpha = jnp.exp(mw - gmax)
        num = num + alpha * aw
        den = den + alpha * sw
    out_ref[...] = (num / den).reshape(1, DIM)


@jax.jit
def kernel(query, keys, values):
    mesh = plsc.VectorSubcoreMesh(core_axis_name="c", subcore_axis_name="s")
    sc_part = pl.kernel(
        _sc_body,
        mesh=mesh,
        compiler_params=pltpu.CompilerParams(needs_layout_passes=False),
        out_type=jax.ShapeDtypeStruct((NW * PW,), jnp.float32),
        scratch_types=[
            pltpu.VMEM((DIM,), jnp.float32),
            pltpu.VMEM((CW,), jnp.float32),
            pltpu.VMEM((CW,), jnp.float32),
            pltpu.VMEM((CW,), jnp.float32),
            pltpu.VMEM((DP,), jnp.float32),
            pltpu.VMEM((DP,), jnp.float32),
            pltpu.VMEM((PW,), jnp.float32),
            pltpu.SemaphoreType.DMA,
            pltpu.SemaphoreType.DMA,
            pltpu.SemaphoreType.DMA,
        ],
    )(query, keys.reshape(-1), values.reshape(-1))

    tc_part = pl.pallas_call(
        _tc_stream_body,
        grid=(TC_NG,),
        in_specs=[
            pl.BlockSpec((1, DIM), lambda g: (0, 0)),
            pl.BlockSpec((TC_BLK, DIM), lambda g: (g, 0)),
            pl.BlockSpec((TC_BLK, DIM), lambda g: (g, 0)),
        ],
        out_specs=pl.BlockSpec((1, PW), lambda g: (0, 0)),
        out_shape=jax.ShapeDtypeStruct((1, PW), jnp.float32),
        scratch_shapes=[
            pltpu.SMEM((1, 1), jnp.float32),
            pltpu.SMEM((1, 1), jnp.float32),
            pltpu.VMEM((1, DIM), jnp.float32),
        ],
    )(query.reshape(1, DIM), keys, values)

    out2d = pl.pallas_call(
        _combine_body,
        out_shape=jax.ShapeDtypeStruct((1, DIM), jnp.float32),
    )(sc_part, tc_part)
    return out2d.reshape(DIM)
